# baseline (speedup 1.0000x reference)
import functools

import jax
import jax.numpy as jnp
from jax import lax
from jax.experimental import pallas as pl
from jax.experimental.pallas import tpu as pltpu
from jax.experimental.pallas import tpu_sc as plsc

HIDDEN = 64
NC = 2
NS = 16
NW = NC * NS
IW = 128
K = 1
NBUF = 4
TBLK = 512


@functools.cache
def _make_transpose_pad(v: int):
    grid = (v + TBLK - 1) // TBLK

    def body(in_ref, out_ref):
        x = in_ref[...]
        out_ref[:, :HIDDEN] = x.T
        out_ref[:, HIDDEN:] = jnp.zeros((TBLK, HIDDEN), jnp.float32)

    return pl.pallas_call(
        body,
        grid=(grid,),
        in_specs=[pl.BlockSpec((HIDDEN, TBLK), lambda i: (0, i))],
        out_specs=pl.BlockSpec((TBLK, 2 * HIDDEN), lambda i: (i, 0)),
        out_shape=jax.ShapeDtypeStruct((v, 2 * HIDDEN), jnp.float32),
    )


@functools.cache
def _make_gather(n_rows: int):
    rows_per_w = n_rows // NW
    chunks = rows_per_w // K
    groups = chunks // NBUF
    chunk = K * IW
    PD = 2

    mesh = plsc.VectorSubcoreMesh(core_axis_name="c", subcore_axis_name="s")

    @functools.partial(
        pl.kernel,
        out_type=jax.ShapeDtypeStruct((n_rows * IW, 2 * HIDDEN), jnp.float32),
        mesh=mesh,
        scratch_types=[
            pltpu.VMEM((rows_per_w, IW), jnp.int32),
            pltpu.VMEM((NBUF, chunk, 2 * HIDDEN), jnp.float32),
            pltpu.SemaphoreType.DMA((NBUF,)),
            pltpu.SemaphoreType.DMA((NBUF,)),
        ],
        compiler_params=pltpu.CompilerParams(use_tc_tiling_on_sc=False),
    )
    def gather_kernel(idx_hbm, table_hbm, out_hbm, idx_v, rows_v, sem_g, sem_o):
        wid = lax.axis_index("s") * NC + lax.axis_index("c")
        row_base = wid * rows_per_w

        pltpu.sync_copy(idx_hbm.at[pl.ds(row_base, rows_per_w)], idx_v)

        def fire_gather(c, b):
            for j in range(K):
                pltpu.async_copy(
                    table_hbm.at[idx_v.at[c * K + j]],
                    rows_v.at[b, pl.ds(j * IW, IW)],
                    sem_g.at[b],
                )

        def wait_gather(c, b):
            for j in range(K):
                pltpu.make_async_copy(
                    table_hbm.at[idx_v.at[c * K + j]],
                    rows_v.at[b, pl.ds(j * IW, IW)],
                    sem_g.at[b],
                ).wait()

        def fire_store(c, b):
            r0 = row_base + c * K
            pltpu.async_copy(
                rows_v.at[b, pl.ds(0, chunk), pl.ds(0, HIDDEN)],
                out_hbm.at[pl.ds(r0 * IW, chunk), pl.ds(0, HIDDEN)],
                sem_o.at[b],
            )

        def wait_store(c, b):
            r0 = row_base + c * K
            pltpu.make_async_copy(
                rows_v.at[b, pl.ds(0, chunk), pl.ds(0, HIDDEN)],
                out_hbm.at[pl.ds(r0 * IW, chunk), pl.ds(0, HIDDEN)],
                sem_o.at[b],
            ).wait()

        for c0 in range(PD):
            fire_gather(c0, c0 % NBUF)

        def body(og, carry):
            for b in range(NBUF):
                c = og * NBUF + b
                cn = c + PD
                bn = (b + PD) % NBUF
                @pl.when(cn < chunks)
                def _():
                    @pl.when(cn >= NBUF)
                    def _():
                        wait_store(cn - NBUF, bn)

                    fire_gather(cn, bn)

                wait_gather(c, b)
                fire_store(c, b)
            return carry

        lax.fori_loop(0, groups, body, 0)

        for b in range(NBUF):
            wait_store(chunks - NBUF + b, b)

    return gather_kernel


def kernel(node_types, table):
    b, n = node_types.shape
    flat = node_types.reshape(-1).astype(jnp.int32)
    total = b * n
    block = NW * NBUF * K * IW
    padded = ((total + block - 1) // block) * block
    if padded != total:
        flat = jnp.pad(flat, (0, padded - total))
    idx2d = flat.reshape(-1, IW)
    tab_pad = _make_transpose_pad(table.shape[0])(jnp.transpose(table))
    out_pad = _make_gather(idx2d.shape[0])(idx2d, tab_pad)
    return out_pad[:total].reshape(b, n, 2 * HIDDEN)[:, :, :HIDDEN]


# speedup vs baseline: 2.1464x; 2.1464x over previous
import functools

import jax
import jax.numpy as jnp
from jax import lax
from jax.experimental import pallas as pl
from jax.experimental.pallas import tpu as pltpu
from jax.experimental.pallas import tpu_sc as plsc

HIDDEN = 64
NC = 2
NS = 16
NW = NC * NS
IW = 128
K = 1
NBUF = 4
TBLK = 4096


@functools.cache
def _make_transpose_pad(v: int):
    grid = (v + TBLK - 1) // TBLK

    def body(in_ref, out_ref):
        x = in_ref[...]
        out_ref[:, :HIDDEN] = x.T
        out_ref[:, HIDDEN:] = jnp.zeros((TBLK, HIDDEN), jnp.float32)

    return pl.pallas_call(
        body,
        grid=(grid,),
        in_specs=[pl.BlockSpec((HIDDEN, TBLK), lambda i: (0, i))],
        out_specs=pl.BlockSpec((TBLK, 2 * HIDDEN), lambda i: (i, 0)),
        out_shape=jax.ShapeDtypeStruct((v, 2 * HIDDEN), jnp.float32),
    )


@functools.cache
def _make_gather(n_rows: int):
    rows_per_w = n_rows // NW
    chunks = rows_per_w // K
    groups = chunks // NBUF
    chunk = K * IW
    PD = 2

    mesh = plsc.VectorSubcoreMesh(core_axis_name="c", subcore_axis_name="s")

    @functools.partial(
        pl.kernel,
        out_type=jax.ShapeDtypeStruct((n_rows * IW, 2 * HIDDEN), jnp.float32),
        mesh=mesh,
        scratch_types=[
            pltpu.VMEM((rows_per_w, IW), jnp.int32),
            pltpu.VMEM((NBUF, chunk, 2 * HIDDEN), jnp.float32),
            pltpu.SemaphoreType.DMA((NBUF,)),
            pltpu.SemaphoreType.DMA((NBUF,)),
        ],
        compiler_params=pltpu.CompilerParams(use_tc_tiling_on_sc=False),
    )
    def gather_kernel(idx_hbm, table_hbm, out_hbm, idx_v, rows_v, sem_g, sem_o):
        wid = lax.axis_index("s") * NC + lax.axis_index("c")
        row_base = wid * rows_per_w

        pltpu.sync_copy(idx_hbm.at[pl.ds(row_base, rows_per_w)], idx_v)

        def fire_gather(c, b):
            for j in range(K):
                pltpu.async_copy(
                    table_hbm.at[idx_v.at[c * K + j]],
                    rows_v.at[b, pl.ds(j * IW, IW)],
                    sem_g.at[b],
                )

        def wait_gather(c, b):
            for j in range(K):
                pltpu.make_async_copy(
                    table_hbm.at[idx_v.at[c * K + j]],
                    rows_v.at[b, pl.ds(j * IW, IW)],
                    sem_g.at[b],
                ).wait()

        def fire_store(c, b):
            r0 = row_base + c * K
            pltpu.async_copy(
                rows_v.at[b, pl.ds(0, chunk), pl.ds(0, HIDDEN)],
                out_hbm.at[pl.ds(r0 * IW, chunk), pl.ds(0, HIDDEN)],
                sem_o.at[b],
            )

        def wait_store(c, b):
            r0 = row_base + c * K
            pltpu.make_async_copy(
                rows_v.at[b, pl.ds(0, chunk), pl.ds(0, HIDDEN)],
                out_hbm.at[pl.ds(r0 * IW, chunk), pl.ds(0, HIDDEN)],
                sem_o.at[b],
            ).wait()

        for c0 in range(PD):
            fire_gather(c0, c0 % NBUF)

        def body(og, carry):
            for b in range(NBUF):
                c = og * NBUF + b
                cn = c + PD
                bn = (b + PD) % NBUF
                @pl.when(cn < chunks)
                def _():
                    @pl.when(cn >= NBUF)
                    def _():
                        wait_store(cn - NBUF, bn)

                    fire_gather(cn, bn)

                wait_gather(c, b)
                fire_store(c, b)
            return carry

        lax.fori_loop(0, groups, body, 0)

        for b in range(NBUF):
            wait_store(chunks - NBUF + b, b)

    return gather_kernel


def kernel(node_types, table):
    b, n = node_types.shape
    flat = node_types.reshape(-1).astype(jnp.int32)
    total = b * n
    block = NW * NBUF * K * IW
    padded = ((total + block - 1) // block) * block
    if padded != total:
        flat = jnp.pad(flat, (0, padded - total))
    idx2d = flat.reshape(-1, IW)
    tab_pad = _make_transpose_pad(table.shape[0])(jnp.transpose(table))
    out_pad = _make_gather(idx2d.shape[0])(idx2d, tab_pad)
    return out_pad[:total].reshape(b, n, 2 * HIDDEN)[:, :, :HIDDEN]


# speedup vs baseline: 2.6345x; 1.2274x over previous
import functools

import jax
import jax.numpy as jnp
from jax import lax
from jax.experimental import pallas as pl
from jax.experimental.pallas import tpu as pltpu
from jax.experimental.pallas import tpu_sc as plsc

HIDDEN = 64
NC = 2
NS = 16
NW = NC * NS
IW = 128
K = 2
NBUF = 4
TBLK = 8192


@functools.cache
def _make_transpose_pad(v: int):
    grid = (v + TBLK - 1) // TBLK

    def body(in_ref, out_ref):
        x = in_ref[...]
        out_ref[:, :HIDDEN] = x.T
        out_ref[:, HIDDEN:] = jnp.zeros((TBLK, HIDDEN), jnp.float32)

    return pl.pallas_call(
        body,
        grid=(grid,),
        in_specs=[pl.BlockSpec((HIDDEN, TBLK), lambda i: (0, i))],
        out_specs=pl.BlockSpec((TBLK, 2 * HIDDEN), lambda i: (i, 0)),
        out_shape=jax.ShapeDtypeStruct((v, 2 * HIDDEN), jnp.float32),
    )


@functools.cache
def _make_gather(n_rows: int):
    rows_per_w = n_rows // NW
    chunks = rows_per_w // K
    groups = chunks // NBUF
    chunk = K * IW
    PD = 2

    mesh = plsc.VectorSubcoreMesh(core_axis_name="c", subcore_axis_name="s")

    @functools.partial(
        pl.kernel,
        out_type=jax.ShapeDtypeStruct((n_rows * IW, 2 * HIDDEN), jnp.float32),
        mesh=mesh,
        scratch_types=[
            pltpu.VMEM((rows_per_w, IW), jnp.int32),
            pltpu.VMEM((NBUF, chunk, HIDDEN), jnp.float32),
            pltpu.SemaphoreType.DMA((NBUF,)),
            pltpu.SemaphoreType.DMA((NBUF,)),
        ],
        compiler_params=pltpu.CompilerParams(use_tc_tiling_on_sc=False),
    )
    def gather_kernel(idx_hbm, table_hbm, out_hbm, idx_v, rows_v, sem_g, sem_o):
        wid = lax.axis_index("s") * NC + lax.axis_index("c")
        row_base = wid * rows_per_w

        pltpu.sync_copy(idx_hbm.at[pl.ds(row_base, rows_per_w)], idx_v)

        def fire_gather(c, b):
            for j in range(K):
                pltpu.async_copy(
                    table_hbm.at[idx_v.at[c * K + j]],
                    rows_v.at[b, pl.ds(j * IW, IW)],
                    sem_g.at[b],
                )

        def wait_gather(c, b):
            for j in range(K):
                pltpu.make_async_copy(
                    table_hbm.at[idx_v.at[c * K + j]],
                    rows_v.at[b, pl.ds(j * IW, IW)],
                    sem_g.at[b],
                ).wait()

        def fire_store(c, b):
            r0 = row_base + c * K
            pltpu.async_copy(
                rows_v.at[b],
                out_hbm.at[pl.ds(r0 * IW, chunk), pl.ds(0, HIDDEN)],
                sem_o.at[b],
            )

        def wait_store(c, b):
            r0 = row_base + c * K
            pltpu.make_async_copy(
                rows_v.at[b],
                out_hbm.at[pl.ds(r0 * IW, chunk), pl.ds(0, HIDDEN)],
                sem_o.at[b],
            ).wait()

        for c0 in range(PD):
            fire_gather(c0, c0 % NBUF)

        def body(og, carry):
            for b in range(NBUF):
                c = og * NBUF + b
                cn = c + PD
                bn = (b + PD) % NBUF
                @pl.when(cn < chunks)
                def _():
                    @pl.when(cn >= NBUF)
                    def _():
                        wait_store(cn - NBUF, bn)

                    fire_gather(cn, bn)

                wait_gather(c, b)
                fire_store(c, b)
            return carry

        lax.fori_loop(0, groups, body, 0)

        for b in range(NBUF):
            wait_store(chunks - NBUF + b, b)

    return gather_kernel


def kernel(node_types, table):
    b, n = node_types.shape
    flat = node_types.reshape(-1).astype(jnp.int32)
    total = b * n
    block = NW * NBUF * K * IW
    padded = ((total + block - 1) // block) * block
    if padded != total:
        flat = jnp.pad(flat, (0, padded - total))
    idx2d = flat.reshape(-1, IW)
    tab_pad = _make_transpose_pad(table.shape[0])(jnp.transpose(table))
    tab2 = tab_pad.reshape(2 * table.shape[0], HIDDEN)
    out_pad = _make_gather(idx2d.shape[0])(idx2d * 2, tab2)
    return out_pad[:total].reshape(b, n, 2 * HIDDEN)[:, :, :HIDDEN]


# speedup vs baseline: 2.7331x; 1.0375x over previous
import functools

import jax
import jax.numpy as jnp
from jax import lax
from jax.experimental import pallas as pl
from jax.experimental.pallas import tpu as pltpu
from jax.experimental.pallas import tpu_sc as plsc

HIDDEN = 64
NC = 2
NS = 16
NW = NC * NS
IW = 128
K = 2
NBUF = 4
TBLK = 16384


@functools.cache
def _make_transpose_pad(v: int):
    grid = (v + TBLK - 1) // TBLK

    def body(in_ref, out_ref):
        x = in_ref[...]
        out_ref[:, :HIDDEN] = x.T
        out_ref[:, HIDDEN:] = jnp.zeros((TBLK, HIDDEN), jnp.float32)

    return pl.pallas_call(
        body,
        grid=(grid,),
        in_specs=[pl.BlockSpec((HIDDEN, TBLK), lambda i: (0, i))],
        out_specs=pl.BlockSpec((TBLK, 2 * HIDDEN), lambda i: (i, 0)),
        out_shape=jax.ShapeDtypeStruct((v, 2 * HIDDEN), jnp.float32),
    )


@functools.cache
def _make_gather(n_rows: int):
    rows_per_w = n_rows // NW
    chunks = rows_per_w // K
    groups = chunks // NBUF
    chunk = K * IW
    PD = 2

    mesh = plsc.VectorSubcoreMesh(core_axis_name="c", subcore_axis_name="s")

    @functools.partial(
        pl.kernel,
        out_type=jax.ShapeDtypeStruct((n_rows * IW, 2 * HIDDEN), jnp.float32),
        mesh=mesh,
        scratch_types=[
            pltpu.VMEM((rows_per_w, IW), jnp.int32),
            pltpu.VMEM((NBUF, chunk, HIDDEN), jnp.float32),
            pltpu.SemaphoreType.DMA((NBUF,)),
            pltpu.SemaphoreType.DMA((NBUF,)),
        ],
        compiler_params=pltpu.CompilerParams(use_tc_tiling_on_sc=False),
    )
    def gather_kernel(idx_hbm, table_hbm, out_hbm, idx_v, rows_v, sem_g, sem_o):
        wid = lax.axis_index("s") * NC + lax.axis_index("c")
        row_base = wid * rows_per_w

        pltpu.sync_copy(idx_hbm.at[pl.ds(row_base, rows_per_w)], idx_v)

        def fire_gather(c, b):
            for j in range(K):
                pltpu.async_copy(
                    table_hbm.at[idx_v.at[c * K + j]],
                    rows_v.at[b, pl.ds(j * IW, IW)],
                    sem_g.at[b],
                )

        def wait_gather(c, b):
            for j in range(K):
                pltpu.make_async_copy(
                    table_hbm.at[idx_v.at[c * K + j]],
                    rows_v.at[b, pl.ds(j * IW, IW)],
                    sem_g.at[b],
                ).wait()

        def fire_store(c, b):
            r0 = row_base + c * K
            pltpu.async_copy(
                rows_v.at[b],
                out_hbm.at[pl.ds(r0 * IW, chunk), pl.ds(0, HIDDEN)],
                sem_o.at[b],
            )

        def wait_store(c, b):
            r0 = row_base + c * K
            pltpu.make_async_copy(
                rows_v.at[b],
                out_hbm.at[pl.ds(r0 * IW, chunk), pl.ds(0, HIDDEN)],
                sem_o.at[b],
            ).wait()

        for c0 in range(PD):
            fire_gather(c0, c0 % NBUF)

        def body(og, carry):
            for b in range(NBUF):
                c = og * NBUF + b
                cn = c + PD
                bn = (b + PD) % NBUF
                @pl.when(cn < chunks)
                def _():
                    @pl.when(cn >= NBUF)
                    def _():
                        wait_store(cn - NBUF, bn)

                    fire_gather(cn, bn)

                wait_gather(c, b)
                fire_store(c, b)
            return carry

        lax.fori_loop(0, groups, body, 0)

        for b in range(NBUF):
            wait_store(chunks - NBUF + b, b)

    return gather_kernel


def kernel(node_types, table):
    b, n = node_types.shape
    flat = node_types.reshape(-1).astype(jnp.int32)
    total = b * n
    block = NW * NBUF * K * IW
    padded = ((total + block - 1) // block) * block
    if padded != total:
        flat = jnp.pad(flat, (0, padded - total))
    idx2d = flat.reshape(-1, IW)
    tab_pad = _make_transpose_pad(table.shape[0])(jnp.transpose(table))
    tab2 = tab_pad.reshape(2 * table.shape[0], HIDDEN)
    out_pad = _make_gather(idx2d.shape[0])(idx2d * 2, tab2)
    return out_pad[:total].reshape(b, n, 2 * HIDDEN)[:, :, :HIDDEN]


# speedup vs baseline: 2.7600x; 1.0098x over previous
import functools

import jax
import jax.numpy as jnp
from jax import lax
from jax.experimental import pallas as pl
from jax.experimental.pallas import tpu as pltpu
from jax.experimental.pallas import tpu_sc as plsc

HIDDEN = 64
NC = 2
NS = 16
NW = NC * NS
IW = 128
K = 2
NBUF = 4
TBLK = 32768


@functools.cache
def _make_transpose_pad(v: int):
    grid = (v + TBLK - 1) // TBLK

    def body(in_ref, out_ref):
        x = in_ref[...]
        out_ref[:, :HIDDEN] = x.T
        out_ref[:, HIDDEN:] = jnp.zeros((TBLK, HIDDEN), jnp.float32)

    return pl.pallas_call(
        body,
        grid=(grid,),
        in_specs=[pl.BlockSpec((HIDDEN, TBLK), lambda i: (0, i))],
        out_specs=pl.BlockSpec((TBLK, 2 * HIDDEN), lambda i: (i, 0)),
        out_shape=jax.ShapeDtypeStruct((v, 2 * HIDDEN), jnp.float32),
    )


@functools.cache
def _make_gather(n_rows: int):
    rows_per_w = n_rows // NW
    chunks = rows_per_w // K
    groups = chunks // NBUF
    chunk = K * IW
    PD = 2

    mesh = plsc.VectorSubcoreMesh(core_axis_name="c", subcore_axis_name="s")

    @functools.partial(
        pl.kernel,
        out_type=jax.ShapeDtypeStruct((n_rows * IW, 2 * HIDDEN), jnp.float32),
        mesh=mesh,
        scratch_types=[
            pltpu.VMEM((rows_per_w, IW), jnp.int32),
            pltpu.VMEM((NBUF, chunk, HIDDEN), jnp.float32),
            pltpu.SemaphoreType.DMA((NBUF,)),
            pltpu.SemaphoreType.DMA((NBUF,)),
        ],
        compiler_params=pltpu.CompilerParams(use_tc_tiling_on_sc=False),
    )
    def gather_kernel(idx_hbm, table_hbm, out_hbm, idx_v, rows_v, sem_g, sem_o):
        wid = lax.axis_index("s") * NC + lax.axis_index("c")
        row_base = wid * rows_per_w

        pltpu.sync_copy(idx_hbm.at[pl.ds(row_base, rows_per_w)], idx_v)

        def fire_gather(c, b):
            for j in range(K):
                pltpu.async_copy(
                    table_hbm.at[idx_v.at[c * K + j]],
                    rows_v.at[b, pl.ds(j * IW, IW)],
                    sem_g.at[b],
                )

        def wait_gather(c, b):
            for j in range(K):
                pltpu.make_async_copy(
                    table_hbm.at[idx_v.at[c * K + j]],
                    rows_v.at[b, pl.ds(j * IW, IW)],
                    sem_g.at[b],
                ).wait()

        def fire_store(c, b):
            r0 = row_base + c * K
            pltpu.async_copy(
                rows_v.at[b],
                out_hbm.at[pl.ds(r0 * IW, chunk), pl.ds(0, HIDDEN)],
                sem_o.at[b],
            )

        def wait_store(c, b):
            r0 = row_base + c * K
            pltpu.make_async_copy(
                rows_v.at[b],
                out_hbm.at[pl.ds(r0 * IW, chunk), pl.ds(0, HIDDEN)],
                sem_o.at[b],
            ).wait()

        for c0 in range(PD):
            fire_gather(c0, c0 % NBUF)

        def body(og, carry):
            for b in range(NBUF):
                c = og * NBUF + b
                cn = c + PD
                bn = (b + PD) % NBUF
                @pl.when(cn < chunks)
                def _():
                    @pl.when(cn >= NBUF)
                    def _():
                        wait_store(cn - NBUF, bn)

                    fire_gather(cn, bn)

                wait_gather(c, b)
                fire_store(c, b)
            return carry

        lax.fori_loop(0, groups, body, 0)

        for b in range(NBUF):
            wait_store(chunks - NBUF + b, b)

    return gather_kernel


def kernel(node_types, table):
    b, n = node_types.shape
    flat = node_types.reshape(-1).astype(jnp.int32)
    total = b * n
    block = NW * NBUF * K * IW
    padded = ((total + block - 1) // block) * block
    if padded != total:
        flat = jnp.pad(flat, (0, padded - total))
    idx2d = flat.reshape(-1, IW)
    tab_pad = _make_transpose_pad(table.shape[0])(jnp.transpose(table))
    tab2 = tab_pad.reshape(2 * table.shape[0], HIDDEN)
    out_pad = _make_gather(idx2d.shape[0])(idx2d * 2, tab2)
    return out_pad[:total].reshape(b, n, 2 * HIDDEN)[:, :, :HIDDEN]
